# baseline (speedup 1.0000x reference)
import functools

import jax
import jax.numpy as jnp
from jax import lax
from jax.experimental import pallas as pl
from jax.experimental.pallas import tpu as pltpu
from jax.experimental.pallas import tpu_sc as plsc

_N = 10000
_D = 128
_E = 320000
_G = 64
_NC = 2
_NS = 16
_EDGES_PER_TILE = _E // (_NC * _NS)
_CHUNK = 80
_NCHUNKS = _EDGES_PER_TILE // _CHUNK
_BCHUNKS = 25
_NBLOCKS = _NCHUNKS // _BCHUNKS
_NPAD = 10240
_ROWS_PER_TILE = _NPAD // _NS


def _sc_agg_body(h_hbm, src_hbm, dst_hbm, zeros_hbm, out_hbm,
                 src_t, dst_t, rows0, rows1, rows2, acc_sh,
                 g0, g1, g2, w0, w1, w2):
  cid = lax.axis_index("c")
  sid = lax.axis_index("s")
  wid = cid * _NS + sid
  row0 = sid * _ROWS_PER_TILE
  rows = (rows0, rows1, rows2)
  gsem = (g0, g1, g2)
  wsem = (w0, w1, w2)
  pltpu.sync_copy(zeros_hbm, acc_sh.at[pl.ds(row0, _ROWS_PER_TILE)])
  plsc.subcore_barrier()

  @pl.loop(0, _NBLOCKS)
  def _(b):
    pltpu.sync_copy(src_hbm.at[wid, b], src_t)
    pltpu.sync_copy(dst_hbm.at[wid, b], dst_t)
    pltpu.async_copy(h_hbm.at[src_t.at[0]], rows[0], gsem[0])

    @pl.loop(0, _BCHUNKS - 1, step=3)
    def _(mb):
      for kk in range(3):
        m = mb + kk
        k1 = (kk + 1) % 3
        pltpu.make_async_copy(h_hbm.at[src_t.at[m]], rows[kk],
                              gsem[kk]).wait()
        pltpu.async_copy(rows[kk], acc_sh.at[dst_t.at[m]], wsem[kk],
                         add=True)

        @pl.when(m >= 2)
        def _():
          pltpu.make_async_copy(rows[k1], acc_sh.at[dst_t.at[m - 2]],
                                wsem[k1]).wait()

        pltpu.async_copy(h_hbm.at[src_t.at[m + 1]], rows[k1], gsem[k1])

    kk = (_BCHUNKS - 1) % 3
    pltpu.make_async_copy(h_hbm.at[src_t.at[_BCHUNKS - 1]], rows[kk],
                          gsem[kk]).wait()
    pltpu.async_copy(rows[kk], acc_sh.at[dst_t.at[_BCHUNKS - 1]],
                     wsem[kk], add=True)
    for m in (_BCHUNKS - 3, _BCHUNKS - 2, _BCHUNKS - 1):
      pltpu.make_async_copy(rows[m % 3], acc_sh.at[dst_t.at[m]],
                            wsem[m % 3]).wait()

  plsc.subcore_barrier()
  pltpu.sync_copy(acc_sh.at[pl.ds(row0, _ROWS_PER_TILE)],
                  out_hbm.at[cid, pl.ds(row0, _ROWS_PER_TILE)])


@jax.jit
def _sc_agg(h, src, dst, zeros):
  mesh = plsc.VectorSubcoreMesh(core_axis_name="c", subcore_axis_name="s")
  k = pl.kernel(
      _sc_agg_body,
      out_type=jax.ShapeDtypeStruct((_NC, _NPAD, _D), jnp.float32),
      mesh=mesh,
      scratch_types=[
          pltpu.VMEM((_BCHUNKS, _CHUNK), jnp.int32),
          pltpu.VMEM((_BCHUNKS, _CHUNK), jnp.int32),
          pltpu.VMEM((_CHUNK, _D), jnp.float32),
          pltpu.VMEM((_CHUNK, _D), jnp.float32),
          pltpu.VMEM((_CHUNK, _D), jnp.float32),
          pltpu.VMEM_SHARED((_NPAD, _D), jnp.float32),
          pltpu.SemaphoreType.DMA,
          pltpu.SemaphoreType.DMA,
          pltpu.SemaphoreType.DMA,
          pltpu.SemaphoreType.DMA,
          pltpu.SemaphoreType.DMA,
          pltpu.SemaphoreType.DMA,
      ],
  )
  nt = _NC * _NS
  return k(h, src.reshape(nt, _NBLOCKS, _BCHUNKS, _CHUNK),
           dst.reshape(nt, _NBLOCKS, _BCHUNKS, _CHUNK), zeros)


def _tc_mlp_body(h_ref, p_ref, scale_ref, w1_ref, b1_ref, w2_ref, b2_ref,
                 out_ref):
  h = h_ref[...]
  z = h * scale_ref[...] + p_ref[0, :_N, :] + p_ref[1, :_N, :]
  a = jnp.dot(z, w1_ref[...], preferred_element_type=jnp.float32,
              precision=lax.Precision.DEFAULT) + b1_ref[...]
  a = jnp.maximum(a, 0.0)
  z2 = jnp.dot(a, w2_ref[...], preferred_element_type=jnp.float32,
               precision=lax.Precision.DEFAULT) + b2_ref[...]
  out_ref[...] = jnp.maximum(z2, 0.0) + h


@jax.jit
def _tc_mlp(h, partials, scale_row, w1, b1r, w2, b2r):
  return pl.pallas_call(
      _tc_mlp_body,
      out_shape=jax.ShapeDtypeStruct((_N, _D), jnp.float32),
  )(h, partials, scale_row, w1, b1r, w2, b2r)


def _tc_head_body(h_ref, batch_ref, gf_ref, wh1a_ref, wh1b_ref, bh1_ref,
                  wh2r_ref, bh2_ref, out_ref):
  ids = lax.broadcasted_iota(jnp.int32, (_G, _N), 0)
  oh = (ids == batch_ref[...]).astype(jnp.float32)
  sums = jnp.dot(oh, h_ref[...], preferred_element_type=jnp.float32,
                 precision=lax.Precision.HIGHEST)
  counts = jnp.sum(oh, axis=1, keepdims=True)
  mean = sums / jnp.maximum(counts, 1.0)
  hid = (jnp.dot(mean, wh1a_ref[...], preferred_element_type=jnp.float32,
                 precision=lax.Precision.DEFAULT)
         + jnp.dot(gf_ref[...], wh1b_ref[...],
                   preferred_element_type=jnp.float32,
                   precision=lax.Precision.DEFAULT)
         + bh1_ref[...])
  hid = jnp.maximum(hid, 0.0)
  out_ref[...] = jnp.sum(hid * wh2r_ref[...], axis=1,
                         keepdims=True) + bh2_ref[...]


@jax.jit
def _tc_head(h, batch_row, gf, wh1a, wh1b, bh1r, wh2r, bh2r):
  return pl.pallas_call(
      _tc_head_body,
      out_shape=jax.ShapeDtypeStruct((_G, 1), jnp.float32),
  )(h, batch_row, gf, wh1a, wh1b, bh1r, wh2r, bh2r)


def kernel(x, global_feats, params, edge_index, batch):
  src = edge_index[0]
  dst = edge_index[1]
  zeros = jnp.zeros((_ROWS_PER_TILE, _D), jnp.float32)
  h = x
  for (eps, w1, b1, w2, b2) in params["convs"]:
    partials = _sc_agg(h, src, dst, zeros)
    scale_row = jnp.full((1, _D), 1.0, jnp.float32) * (1.0 + eps)
    h = _tc_mlp(h, partials, scale_row, w1, b1.reshape(1, _D), w2,
                b2.reshape(1, _D))
  wh1, bh1, wh2, bh2 = params["head"]
  logits2d = _tc_head(h, batch.reshape(1, _N), global_feats,
                      wh1[:_D], wh1[_D:], bh1.reshape(1, _D),
                      wh2.reshape(1, _D), bh2.reshape(1, 1))
  return logits2d.reshape(_G)


# speedup vs baseline: 1.4014x; 1.4014x over previous
import functools

import jax
import jax.numpy as jnp
from jax import lax
from jax.experimental import pallas as pl
from jax.experimental.pallas import tpu as pltpu
from jax.experimental.pallas import tpu_sc as plsc

_N = 10000
_D = 128
_E = 320000
_G = 64
_NC = 2
_NS = 16
_EDGES_PER_TILE = _E // (_NC * _NS)
_CHUNK = 80
_NCHUNKS = _EDGES_PER_TILE // _CHUNK
_BCHUNKS = 25
_NBLOCKS = _NCHUNKS // _BCHUNKS
_NPAD = 10240
_ROWS_PER_TILE = _NPAD // _NS


def _sc_agg_body(h_hbm, src_hbm, dst_hbm, zeros_hbm, out_hbm,
                 src_t, dst_t, rows0, rows1, rows2, acc_sh,
                 g0, g1, g2, w0, w1, w2):
  cid = lax.axis_index("c")
  sid = lax.axis_index("s")
  wid = cid * _NS + sid
  row0 = sid * _ROWS_PER_TILE
  rows = (rows0, rows1, rows2)
  gsem = (g0, g1, g2)
  wsem = (w0, w1, w2)
  pltpu.sync_copy(zeros_hbm, acc_sh.at[pl.ds(row0, _ROWS_PER_TILE)])
  plsc.subcore_barrier()

  @pl.loop(0, _NBLOCKS)
  def _(b):
    pltpu.sync_copy(src_hbm.at[wid, b], src_t)
    pltpu.sync_copy(dst_hbm.at[wid, b], dst_t)
    pltpu.async_copy(h_hbm.at[src_t.at[0]], rows[0], gsem[0])
    pltpu.async_copy(h_hbm.at[src_t.at[1]], rows[1], gsem[1])

    @pl.loop(0, _BCHUNKS - 4, step=3)
    def _(mb):
      for kk in range(3):
        m = mb + kk
        k2 = (kk + 2) % 3
        pltpu.make_async_copy(h_hbm.at[src_t.at[m]], rows[kk],
                              gsem[kk]).wait()
        pltpu.async_copy(rows[kk], acc_sh.at[dst_t.at[m]], wsem[kk],
                         add=True)

        @pl.when(m >= 1)
        def _():
          pltpu.make_async_copy(rows[k2], acc_sh.at[dst_t.at[m - 1]],
                                wsem[k2]).wait()

        pltpu.async_copy(h_hbm.at[src_t.at[m + 2]], rows[k2], gsem[k2])

    for m in range(_BCHUNKS - 4, _BCHUNKS):
      kk = m % 3
      k2 = (kk + 2) % 3
      pltpu.make_async_copy(h_hbm.at[src_t.at[m]], rows[kk],
                            gsem[kk]).wait()
      pltpu.async_copy(rows[kk], acc_sh.at[dst_t.at[m]], wsem[kk],
                       add=True)
      pltpu.make_async_copy(rows[k2], acc_sh.at[dst_t.at[m - 1]],
                            wsem[k2]).wait()
      if m + 2 < _BCHUNKS:
        pltpu.async_copy(h_hbm.at[src_t.at[m + 2]], rows[k2], gsem[k2])
    pltpu.make_async_copy(rows[(_BCHUNKS - 1) % 3],
                          acc_sh.at[dst_t.at[_BCHUNKS - 1]],
                          wsem[(_BCHUNKS - 1) % 3]).wait()

  plsc.subcore_barrier()
  pltpu.sync_copy(acc_sh.at[pl.ds(row0, _ROWS_PER_TILE)],
                  out_hbm.at[cid, pl.ds(row0, _ROWS_PER_TILE)])


@jax.jit
def _sc_agg(h, src, dst, zeros):
  mesh = plsc.VectorSubcoreMesh(core_axis_name="c", subcore_axis_name="s")
  k = pl.kernel(
      _sc_agg_body,
      out_type=jax.ShapeDtypeStruct((_NC, _NPAD, _D), jnp.float32),
      mesh=mesh,
      scratch_types=[
          pltpu.VMEM((_BCHUNKS, _CHUNK), jnp.int32),
          pltpu.VMEM((_BCHUNKS, _CHUNK), jnp.int32),
          pltpu.VMEM((_CHUNK, _D), jnp.float32),
          pltpu.VMEM((_CHUNK, _D), jnp.float32),
          pltpu.VMEM((_CHUNK, _D), jnp.float32),
          pltpu.VMEM_SHARED((_NPAD, _D), jnp.float32),
          pltpu.SemaphoreType.DMA,
          pltpu.SemaphoreType.DMA,
          pltpu.SemaphoreType.DMA,
          pltpu.SemaphoreType.DMA,
          pltpu.SemaphoreType.DMA,
          pltpu.SemaphoreType.DMA,
      ],
  )
  nt = _NC * _NS
  return k(h, src.reshape(nt, _NBLOCKS, _BCHUNKS, _CHUNK),
           dst.reshape(nt, _NBLOCKS, _BCHUNKS, _CHUNK), zeros)


def _tc_mlp_body(h_ref, p_ref, scale_ref, w1_ref, b1_ref, w2_ref, b2_ref,
                 out_ref):
  h = h_ref[...]
  z = h * scale_ref[...] + p_ref[0, :_N, :] + p_ref[1, :_N, :]
  a = jnp.dot(z, w1_ref[...], preferred_element_type=jnp.float32,
              precision=lax.Precision.DEFAULT) + b1_ref[...]
  a = jnp.maximum(a, 0.0)
  z2 = jnp.dot(a, w2_ref[...], preferred_element_type=jnp.float32,
               precision=lax.Precision.DEFAULT) + b2_ref[...]
  out_ref[...] = jnp.maximum(z2, 0.0) + h


@jax.jit
def _tc_mlp(h, partials, scale_row, w1, b1r, w2, b2r):
  return pl.pallas_call(
      _tc_mlp_body,
      out_shape=jax.ShapeDtypeStruct((_N, _D), jnp.float32),
  )(h, partials, scale_row, w1, b1r, w2, b2r)


def _tc_head_body(h_ref, batch_ref, gf_ref, wh1a_ref, wh1b_ref, bh1_ref,
                  wh2r_ref, bh2_ref, out_ref):
  ids = lax.broadcasted_iota(jnp.int32, (_G, _N), 0)
  oh = (ids == batch_ref[...]).astype(jnp.float32)
  sums = jnp.dot(oh, h_ref[...], preferred_element_type=jnp.float32,
                 precision=lax.Precision.HIGHEST)
  counts = jnp.sum(oh, axis=1, keepdims=True)
  mean = sums / jnp.maximum(counts, 1.0)
  hid = (jnp.dot(mean, wh1a_ref[...], preferred_element_type=jnp.float32,
                 precision=lax.Precision.DEFAULT)
         + jnp.dot(gf_ref[...], wh1b_ref[...],
                   preferred_element_type=jnp.float32,
                   precision=lax.Precision.DEFAULT)
         + bh1_ref[...])
  hid = jnp.maximum(hid, 0.0)
  out_ref[...] = jnp.sum(hid * wh2r_ref[...], axis=1,
                         keepdims=True) + bh2_ref[...]


@jax.jit
def _tc_head(h, batch_row, gf, wh1a, wh1b, bh1r, wh2r, bh2r):
  return pl.pallas_call(
      _tc_head_body,
      out_shape=jax.ShapeDtypeStruct((_G, 1), jnp.float32),
  )(h, batch_row, gf, wh1a, wh1b, bh1r, wh2r, bh2r)


def kernel(x, global_feats, params, edge_index, batch):
  src = edge_index[0]
  dst = edge_index[1]
  zeros = jnp.zeros((_ROWS_PER_TILE, _D), jnp.float32)
  h = x
  for (eps, w1, b1, w2, b2) in params["convs"]:
    partials = _sc_agg(h, src, dst, zeros)
    scale_row = jnp.full((1, _D), 1.0, jnp.float32) * (1.0 + eps)
    h = _tc_mlp(h, partials, scale_row, w1, b1.reshape(1, _D), w2,
                b2.reshape(1, _D))
  wh1, bh1, wh2, bh2 = params["head"]
  logits2d = _tc_head(h, batch.reshape(1, _N), global_feats,
                      wh1[:_D], wh1[_D:], bh1.reshape(1, _D),
                      wh2.reshape(1, _D), bh2.reshape(1, 1))
  return logits2d.reshape(_G)
